# Initial kernel scaffold; baseline (speedup 1.0000x reference)
import jax
import jax.numpy as jnp
from jax.experimental import pallas as pl


def kernel(x, nh_idx, coords_rel, phis, dists, sigma, W_lin):
    raise NotImplementedError("write your pallas kernel here")


# speedup vs baseline: 1.0799x; 1.0799x over previous
import functools
import math

import jax
import jax.numpy as jnp
from jax import lax
from jax.experimental import pallas as pl
from jax.experimental.pallas import tpu as pltpu
from jax.experimental.pallas import tpu_sc as plsc

N = 10000
K = 32
A_IN = 128
A_OUT = 128
NPHI = 4
NDIST = 4
PD = NPHI * NDIST
SIG_PHI = 2.0 * math.pi / NPHI

NC = 2
NS = 16
NW = NC * NS
NK = N * K
NK_PER_W = NK // NW
R = 80
CPW = NK_PER_W // R


def _sc_gather_body(x_hbm, idx_hbm, out_hbm, idx_v, rows_v, sem):
    wid = lax.axis_index("s") * NC + lax.axis_index("c")
    base = wid * NK_PER_W

    def chunk(c, _):
        st = base + c * R
        pltpu.sync_copy(idx_hbm.at[pl.ds(st, R)], idx_v)
        cp = pltpu.make_async_copy(x_hbm.at[idx_v], rows_v, sem)
        cp.start()
        cp.wait()
        pltpu.sync_copy(rows_v, out_hbm.at[pl.ds(st, R)])
        return 0

    lax.fori_loop(0, CPW, chunk, 0)


@functools.lru_cache(maxsize=None)
def _get_sc_gather():
    return pl.kernel(
        _sc_gather_body,
        out_type=jax.ShapeDtypeStruct((NK, A_IN), jnp.float32),
        mesh=plsc.VectorSubcoreMesh(core_axis_name="c", subcore_axis_name="s",
                                    num_cores=NC, num_subcores=NS),
        scratch_types=[
            pltpu.VMEM((R,), jnp.int32),
            pltpu.VMEM((R, A_IN), jnp.float32),
            pltpu.SemaphoreType.DMA,
        ],
    )


BN = 200


def _tc_body(phis_s, dists_s, sigma_s, dx_ref, dy_ref, g_ref, wl_ref,
             out_ref):
    dx = dx_ref[...]
    dy = dy_ref[...]
    r = jnp.sqrt(dx * dx + dy * dy + 1e-10)
    theta = jnp.arctan2(dy, dx)
    s = jnp.maximum(sigma_s[0], 1e-10)
    angs = []
    for p in range(NPHI):
        t = theta - phis_s[p] + math.pi
        t = t - jnp.floor(t / (2.0 * math.pi)) * (2.0 * math.pi)
        dphi = t - math.pi
        angs.append(jnp.exp(-0.5 * (dphi / SIG_PHI) ** 2))
    rads = []
    for d in range(NDIST):
        dr = (r - dists_s[d]) / s
        rads.append(jnp.exp(-0.5 * dr * dr))
    g = g_ref[...]
    wl = wl_ref[...]
    for p in range(NPHI):
        for d in range(NDIST):
            w = angs[p] * rads[d]
            den = jnp.sum(w, axis=1, keepdims=True) + 1e-10
            wn = w / den
            agg = jnp.sum(wn[:, :, None] * g, axis=1)
            out_ref[:, p * NDIST + d, :] = jnp.dot(
                agg, wl, preferred_element_type=jnp.float32)


_tc_agg = pl.pallas_call(
    _tc_body,
    grid=(N // BN,),
    in_specs=[
        pl.BlockSpec(memory_space=pltpu.SMEM),
        pl.BlockSpec(memory_space=pltpu.SMEM),
        pl.BlockSpec(memory_space=pltpu.SMEM),
        pl.BlockSpec((BN, K), lambda i: (i, 0)),
        pl.BlockSpec((BN, K), lambda i: (i, 0)),
        pl.BlockSpec((BN, K, A_IN), lambda i: (i, 0, 0)),
        pl.BlockSpec((A_IN, A_OUT), lambda i: (0, 0)),
    ],
    out_specs=pl.BlockSpec((BN, PD, A_OUT), lambda i: (i, 0, 0)),
    out_shape=jax.ShapeDtypeStruct((N, PD, A_OUT), jnp.float32),
)


def kernel(x, nh_idx, coords_rel, phis, dists, sigma, W_lin):
    x2d = x.reshape(N, A_IN)
    flat_idx = nh_idx.reshape(NK).astype(jnp.int32)
    g = _get_sc_gather()(x2d, flat_idx)
    dx = coords_rel[:, :, 0]
    dy = coords_rel[:, :, 1]
    out = _tc_agg(phis, dists, sigma, dx, dy, g.reshape(N, K, A_IN), W_lin)
    return out.reshape(1, N, NPHI, NDIST, 1, A_OUT)


# speedup vs baseline: 1.4837x; 1.3738x over previous
import functools
import math

import jax
import jax.numpy as jnp
from jax import lax
from jax.experimental import pallas as pl
from jax.experimental.pallas import tpu as pltpu
from jax.experimental.pallas import tpu_sc as plsc

N = 10000
K = 32
A_IN = 128
A_OUT = 128
NPHI = 4
NDIST = 4
PD = NPHI * NDIST
SIG_PHI = 2.0 * math.pi / NPHI

NC = 2
NS = 16
NW = NC * NS
NK = N * K
NK_PER_W = NK // NW
R = 80
AW = A_OUT
CPW = NK_PER_W // R


def _sc_gather_body(y_hbm, idx_hbm, out_hbm, idx_v, rows_v, sem):
    wid = lax.axis_index("s") * NC + lax.axis_index("c")
    base = wid * NK_PER_W

    def chunk(c, _):
        st = base + c * R
        pltpu.sync_copy(idx_hbm.at[pl.ds(st, R)], idx_v)
        cp = pltpu.make_async_copy(y_hbm.at[idx_v], rows_v, sem)
        cp.start()
        cp.wait()
        pltpu.sync_copy(rows_v, out_hbm.at[pl.ds(st, R)])
        return 0

    lax.fori_loop(0, CPW, chunk, 0)


@functools.lru_cache(maxsize=None)
def _get_sc_gather():
    return pl.kernel(
        _sc_gather_body,
        out_type=jax.ShapeDtypeStruct((NK, AW), jnp.int32),
        mesh=plsc.VectorSubcoreMesh(core_axis_name="c", subcore_axis_name="s",
                                    num_cores=NC, num_subcores=NS),
        scratch_types=[
            pltpu.VMEM((R,), jnp.int32),
            pltpu.VMEM((R, AW), jnp.int32),
            pltpu.SemaphoreType.DMA,
        ],
    )


NPRE = 2000


def _prew_body(x_ref, w_ref, y_ref):
    y_ref[...] = jnp.dot(x_ref[...], w_ref[...],
                         preferred_element_type=jnp.float32)


_prew = pl.pallas_call(
    _prew_body,
    grid=(N // NPRE,),
    in_specs=[
        pl.BlockSpec((NPRE, A_IN), lambda i: (i, 0)),
        pl.BlockSpec((A_IN, A_OUT), lambda i: (0, 0)),
    ],
    out_specs=pl.BlockSpec((NPRE, A_OUT), lambda i: (i, 0)),
    out_shape=jax.ShapeDtypeStruct((N, A_OUT), jnp.float32),
)


BN = 200
GRP = 8
NG = BN // GRP


def _tc_body(phis_s, dists_s, sigma_s, dx_ref, dy_ref, g_ref, out_ref):
    dx = dx_ref[...]
    dy = dy_ref[...]
    r = jnp.sqrt(dx * dx + dy * dy + 1e-10)
    theta = jnp.arctan2(dy, dx)
    s = jnp.maximum(sigma_s[0], 1e-10)
    angs = []
    for p in range(NPHI):
        t = theta - phis_s[p] + math.pi
        t = t - jnp.floor(t / (2.0 * math.pi)) * (2.0 * math.pi)
        dphi = t - math.pi
        angs.append(jnp.exp(-0.5 * (dphi / SIG_PHI) ** 2))
    rads = []
    for d in range(NDIST):
        dr = (r - dists_s[d]) / s
        rads.append(jnp.exp(-0.5 * dr * dr))
    ang_t = jnp.stack(angs, axis=1)
    rad_t = jnp.stack(rads, axis=1)
    w4 = ang_t[:, :, None, :] * rad_t[:, None, :, :]
    w_t = w4.reshape(BN, PD, K)
    den = jnp.sum(w_t, axis=2, keepdims=True) + 1e-10
    wn = w_t / den

    ii = lax.broadcasted_iota(jnp.int32, (GRP, GRP), 0)
    jj = lax.broadcasted_iota(jnp.int32, (GRP, GRP), 1)
    eye = jnp.where(ii == jj, 1.0, 0.0).astype(jnp.float32)
    lhs5 = wn.reshape(NG, GRP, PD, 1, K) * eye.reshape(1, GRP, 1, GRP, 1)
    lhs = lhs5.reshape(NG, GRP * PD, GRP * K)
    rhs = g_ref[...].reshape(NG, GRP * K, A_OUT)
    outs = []
    for g in range(NG):
        outs.append(jnp.dot(lhs[g], rhs[g],
                            preferred_element_type=jnp.float32))
    out = jnp.stack(outs, axis=0)
    out_ref[...] = out.reshape(BN, PD, A_OUT)


_tc_agg = pl.pallas_call(
    _tc_body,
    grid=(N // BN,),
    in_specs=[
        pl.BlockSpec(memory_space=pltpu.SMEM),
        pl.BlockSpec(memory_space=pltpu.SMEM),
        pl.BlockSpec(memory_space=pltpu.SMEM),
        pl.BlockSpec((BN, K), lambda i: (i, 0)),
        pl.BlockSpec((BN, K), lambda i: (i, 0)),
        pl.BlockSpec((BN, K, A_OUT), lambda i: (i, 0, 0)),
    ],
    out_specs=pl.BlockSpec((BN, PD, A_OUT), lambda i: (i, 0, 0)),
    out_shape=jax.ShapeDtypeStruct((N, PD, A_OUT), jnp.float32),
)


def kernel(x, nh_idx, coords_rel, phis, dists, sigma, W_lin):
    x2d = x.reshape(N, A_IN)
    y = _prew(x2d, W_lin)
    y_pack = lax.bitcast_convert_type(y, jnp.int32)
    flat_idx = nh_idx.reshape(NK).astype(jnp.int32)
    g = _get_sc_gather()(y_pack, flat_idx)
    g_bf = lax.bitcast_convert_type(g, jnp.float32).reshape(N, K, A_OUT)
    dx = coords_rel[:, :, 0]
    dy = coords_rel[:, :, 1]
    out = _tc_agg(phis, dists, sigma, dx, dy, g_bf)
    return out.reshape(1, N, NPHI, NDIST, 1, A_OUT)


# speedup vs baseline: 1.6760x; 1.1296x over previous
import functools
import math

import jax
import jax.numpy as jnp
from jax import lax
from jax.experimental import pallas as pl
from jax.experimental.pallas import tpu as pltpu
from jax.experimental.pallas import tpu_sc as plsc

N = 10000
K = 32
A_IN = 128
A_OUT = 128
NPHI = 4
NDIST = 4
PD = NPHI * NDIST
SIG_PHI = 2.0 * math.pi / NPHI

NC = 2
NS = 16
NW = NC * NS
NK = N * K
NK_PER_W = NK // NW
R = 200
AW = A_OUT
CPW = NK_PER_W // R


def _sc_gather_body(y_hbm, idx_hbm, out_hbm, idx_v, rows_v, sem):
    wid = lax.axis_index("s") * NC + lax.axis_index("c")
    base = wid * NK_PER_W

    def chunk(c, _):
        st = base + c * R
        pltpu.sync_copy(idx_hbm.at[pl.ds(st, R)], idx_v)
        cp = pltpu.make_async_copy(y_hbm.at[idx_v], rows_v, sem)
        cp.start()
        cp.wait()
        pltpu.sync_copy(rows_v, out_hbm.at[pl.ds(st, R)])
        return 0

    lax.fori_loop(0, CPW, chunk, 0)


@functools.lru_cache(maxsize=None)
def _get_sc_gather():
    return pl.kernel(
        _sc_gather_body,
        out_type=jax.ShapeDtypeStruct((NK, AW), jnp.int32),
        mesh=plsc.VectorSubcoreMesh(core_axis_name="c", subcore_axis_name="s",
                                    num_cores=NC, num_subcores=NS),
        scratch_types=[
            pltpu.VMEM((R,), jnp.int32),
            pltpu.VMEM((R, AW), jnp.int32),
            pltpu.SemaphoreType.DMA,
        ],
    )


NPRE = 2000


def _prew_body(x_ref, w_ref, y_ref):
    y_ref[...] = jnp.dot(x_ref[...], w_ref[...],
                         preferred_element_type=jnp.float32)


_prew = pl.pallas_call(
    _prew_body,
    grid=(N // NPRE,),
    in_specs=[
        pl.BlockSpec((NPRE, A_IN), lambda i: (i, 0)),
        pl.BlockSpec((A_IN, A_OUT), lambda i: (0, 0)),
    ],
    out_specs=pl.BlockSpec((NPRE, A_OUT), lambda i: (i, 0)),
    out_shape=jax.ShapeDtypeStruct((N, A_OUT), jnp.float32),
)


BN = 200
GRP = 8
NG = BN // GRP


def _tc_body(phis_s, dists_s, sigma_s, dx_ref, dy_ref, g_ref, out_ref):
    dx = dx_ref[...]
    dy = dy_ref[...]
    r = jnp.sqrt(dx * dx + dy * dy + 1e-10)
    theta = jnp.arctan2(dy, dx)
    s = jnp.maximum(sigma_s[0], 1e-10)
    angs = []
    for p in range(NPHI):
        t = theta - phis_s[p] + math.pi
        t = t - jnp.floor(t / (2.0 * math.pi)) * (2.0 * math.pi)
        dphi = t - math.pi
        angs.append(jnp.exp(-0.5 * (dphi / SIG_PHI) ** 2))
    rads = []
    for d in range(NDIST):
        dr = (r - dists_s[d]) / s
        rads.append(jnp.exp(-0.5 * dr * dr))
    ang_t = jnp.stack(angs, axis=1)
    rad_t = jnp.stack(rads, axis=1)
    w4 = ang_t[:, :, None, :] * rad_t[:, None, :, :]
    w_t = w4.reshape(BN, PD, K)
    den = jnp.sum(w_t, axis=2, keepdims=True) + 1e-10
    wn = w_t / den

    ii = lax.broadcasted_iota(jnp.int32, (GRP, GRP), 0)
    jj = lax.broadcasted_iota(jnp.int32, (GRP, GRP), 1)
    eye = jnp.where(ii == jj, 1.0, 0.0).astype(jnp.float32)
    lhs5 = wn.reshape(NG, GRP, PD, 1, K) * eye.reshape(1, GRP, 1, GRP, 1)
    lhs = lhs5.reshape(NG, GRP * PD, GRP * K)
    rhs = g_ref[...].reshape(NG, GRP * K, A_OUT)
    outs = []
    for g in range(NG):
        outs.append(jnp.dot(lhs[g], rhs[g],
                            preferred_element_type=jnp.float32))
    out = jnp.stack(outs, axis=0)
    out_ref[...] = out.reshape(BN, PD, A_OUT)


_tc_agg = pl.pallas_call(
    _tc_body,
    grid=(N // BN,),
    in_specs=[
        pl.BlockSpec(memory_space=pltpu.SMEM),
        pl.BlockSpec(memory_space=pltpu.SMEM),
        pl.BlockSpec(memory_space=pltpu.SMEM),
        pl.BlockSpec((BN, K), lambda i: (i, 0)),
        pl.BlockSpec((BN, K), lambda i: (i, 0)),
        pl.BlockSpec((BN, K, A_OUT), lambda i: (i, 0, 0)),
    ],
    out_specs=pl.BlockSpec((BN, PD, A_OUT), lambda i: (i, 0, 0)),
    out_shape=jax.ShapeDtypeStruct((N, PD, A_OUT), jnp.float32),
)


def kernel(x, nh_idx, coords_rel, phis, dists, sigma, W_lin):
    x2d = x.reshape(N, A_IN)
    y = _prew(x2d, W_lin)
    y_pack = lax.bitcast_convert_type(y, jnp.int32)
    flat_idx = nh_idx.reshape(NK).astype(jnp.int32)
    g = _get_sc_gather()(y_pack, flat_idx)
    g_bf = lax.bitcast_convert_type(g, jnp.float32).reshape(N, K, A_OUT)
    dx = coords_rel[:, :, 0]
    dy = coords_rel[:, :, 1]
    out = _tc_agg(phis, dists, sigma, dx, dy, g_bf)
    return out.reshape(1, N, NPHI, NDIST, 1, A_OUT)


# speedup vs baseline: 1.7675x; 1.0546x over previous
import functools
import math

import jax
import jax.numpy as jnp
from jax import lax
from jax.experimental import pallas as pl
from jax.experimental.pallas import tpu as pltpu
from jax.experimental.pallas import tpu_sc as plsc

N = 10000
K = 32
A_IN = 128
A_OUT = 128
NPHI = 4
NDIST = 4
PD = NPHI * NDIST
SIG_PHI = 2.0 * math.pi / NPHI

NC = 2
NS = 16
NW = NC * NS
NK = N * K
NK_PER_W = NK // NW
R = 400
AW = A_OUT
CPW = NK_PER_W // R


def _sc_gather_body(y_hbm, idx_hbm, out_hbm, idx_v, rows_v, sem):
    wid = lax.axis_index("s") * NC + lax.axis_index("c")
    base = wid * NK_PER_W

    def chunk(c, _):
        st = base + c * R
        pltpu.sync_copy(idx_hbm.at[pl.ds(st, R)], idx_v)
        cp = pltpu.make_async_copy(y_hbm.at[idx_v], rows_v, sem)
        cp.start()
        cp.wait()
        pltpu.sync_copy(rows_v, out_hbm.at[pl.ds(st, R)])
        return 0

    lax.fori_loop(0, CPW, chunk, 0)


@functools.lru_cache(maxsize=None)
def _get_sc_gather():
    return pl.kernel(
        _sc_gather_body,
        out_type=jax.ShapeDtypeStruct((NK, AW), jnp.int32),
        mesh=plsc.VectorSubcoreMesh(core_axis_name="c", subcore_axis_name="s",
                                    num_cores=NC, num_subcores=NS),
        scratch_types=[
            pltpu.VMEM((R,), jnp.int32),
            pltpu.VMEM((R, AW), jnp.int32),
            pltpu.SemaphoreType.DMA,
        ],
    )


NPRE = 2000


def _prew_body(x_ref, w_ref, y_ref):
    y_ref[...] = jnp.dot(x_ref[...], w_ref[...],
                         preferred_element_type=jnp.float32)


_prew = pl.pallas_call(
    _prew_body,
    grid=(N // NPRE,),
    in_specs=[
        pl.BlockSpec((NPRE, A_IN), lambda i: (i, 0)),
        pl.BlockSpec((A_IN, A_OUT), lambda i: (0, 0)),
    ],
    out_specs=pl.BlockSpec((NPRE, A_OUT), lambda i: (i, 0)),
    out_shape=jax.ShapeDtypeStruct((N, A_OUT), jnp.float32),
)


BN = 200
GRP = 8
NG = BN // GRP


def _tc_body(phis_s, dists_s, sigma_s, dx_ref, dy_ref, g_ref, out_ref):
    dx = dx_ref[...]
    dy = dy_ref[...]
    r = jnp.sqrt(dx * dx + dy * dy + 1e-10)
    theta = jnp.arctan2(dy, dx)
    s = jnp.maximum(sigma_s[0], 1e-10)
    angs = []
    for p in range(NPHI):
        t = theta - phis_s[p] + math.pi
        t = t - jnp.floor(t / (2.0 * math.pi)) * (2.0 * math.pi)
        dphi = t - math.pi
        angs.append(jnp.exp(-0.5 * (dphi / SIG_PHI) ** 2))
    rads = []
    for d in range(NDIST):
        dr = (r - dists_s[d]) / s
        rads.append(jnp.exp(-0.5 * dr * dr))
    ang_t = jnp.stack(angs, axis=1)
    rad_t = jnp.stack(rads, axis=1)
    w4 = ang_t[:, :, None, :] * rad_t[:, None, :, :]
    w_t = w4.reshape(BN, PD, K)
    den = jnp.sum(w_t, axis=2, keepdims=True) + 1e-10
    wn = w_t / den

    ii = lax.broadcasted_iota(jnp.int32, (GRP, GRP), 0)
    jj = lax.broadcasted_iota(jnp.int32, (GRP, GRP), 1)
    eye = jnp.where(ii == jj, 1.0, 0.0).astype(jnp.float32)
    lhs5 = wn.reshape(NG, GRP, PD, 1, K) * eye.reshape(1, GRP, 1, GRP, 1)
    lhs = lhs5.reshape(NG, GRP * PD, GRP * K)
    rhs = g_ref[...].reshape(NG, GRP * K, A_OUT)
    outs = []
    for g in range(NG):
        outs.append(jnp.dot(lhs[g], rhs[g],
                            preferred_element_type=jnp.float32))
    out = jnp.stack(outs, axis=0)
    out_ref[...] = out.reshape(BN, PD, A_OUT)


_tc_agg = pl.pallas_call(
    _tc_body,
    grid=(N // BN,),
    in_specs=[
        pl.BlockSpec(memory_space=pltpu.SMEM),
        pl.BlockSpec(memory_space=pltpu.SMEM),
        pl.BlockSpec(memory_space=pltpu.SMEM),
        pl.BlockSpec((BN, K), lambda i: (i, 0)),
        pl.BlockSpec((BN, K), lambda i: (i, 0)),
        pl.BlockSpec((BN, K, A_OUT), lambda i: (i, 0, 0)),
    ],
    out_specs=pl.BlockSpec((BN, PD, A_OUT), lambda i: (i, 0, 0)),
    out_shape=jax.ShapeDtypeStruct((N, PD, A_OUT), jnp.float32),
)


def kernel(x, nh_idx, coords_rel, phis, dists, sigma, W_lin):
    x2d = x.reshape(N, A_IN)
    y = _prew(x2d, W_lin)
    y_pack = lax.bitcast_convert_type(y, jnp.int32)
    flat_idx = nh_idx.reshape(NK).astype(jnp.int32)
    g = _get_sc_gather()(y_pack, flat_idx)
    g_bf = lax.bitcast_convert_type(g, jnp.float32).reshape(N, K, A_OUT)
    dx = coords_rel[:, :, 0]
    dy = coords_rel[:, :, 1]
    out = _tc_agg(phis, dists, sigma, dx, dy, g_bf)
    return out.reshape(1, N, NPHI, NDIST, 1, A_OUT)


# speedup vs baseline: 1.8368x; 1.0392x over previous
import functools
import math

import jax
import jax.numpy as jnp
from jax import lax
from jax.experimental import pallas as pl
from jax.experimental.pallas import tpu as pltpu
from jax.experimental.pallas import tpu_sc as plsc

N = 10000
K = 32
A_IN = 128
A_OUT = 128
NPHI = 4
NDIST = 4
PD = NPHI * NDIST
SIG_PHI = 2.0 * math.pi / NPHI

NC = 2
NS = 16
NW = NC * NS
NK = N * K
NK_PER_W = NK // NW
R = 1000
AW = A_OUT
CPW = NK_PER_W // R


def _sc_gather_body(y_hbm, idx_hbm, out_hbm, idx_v, rows_v, sem):
    wid = lax.axis_index("s") * NC + lax.axis_index("c")
    base = wid * NK_PER_W

    def chunk(c, _):
        st = base + c * R
        pltpu.sync_copy(idx_hbm.at[pl.ds(st, R)], idx_v)
        cp = pltpu.make_async_copy(y_hbm.at[idx_v], rows_v, sem)
        cp.start()
        cp.wait()
        pltpu.sync_copy(rows_v, out_hbm.at[pl.ds(st, R)])
        return 0

    lax.fori_loop(0, CPW, chunk, 0)


@functools.lru_cache(maxsize=None)
def _get_sc_gather():
    return pl.kernel(
        _sc_gather_body,
        out_type=jax.ShapeDtypeStruct((NK, AW), jnp.int32),
        mesh=plsc.VectorSubcoreMesh(core_axis_name="c", subcore_axis_name="s",
                                    num_cores=NC, num_subcores=NS),
        scratch_types=[
            pltpu.VMEM((R,), jnp.int32),
            pltpu.VMEM((R, AW), jnp.int32),
            pltpu.SemaphoreType.DMA,
        ],
    )


NPRE = 2000


def _prew_body(x_ref, w_ref, y_ref):
    y_ref[...] = jnp.dot(x_ref[...], w_ref[...],
                         preferred_element_type=jnp.float32)


_prew = pl.pallas_call(
    _prew_body,
    grid=(N // NPRE,),
    in_specs=[
        pl.BlockSpec((NPRE, A_IN), lambda i: (i, 0)),
        pl.BlockSpec((A_IN, A_OUT), lambda i: (0, 0)),
    ],
    out_specs=pl.BlockSpec((NPRE, A_OUT), lambda i: (i, 0)),
    out_shape=jax.ShapeDtypeStruct((N, A_OUT), jnp.float32),
)


BN = 200
GRP = 8
NG = BN // GRP


def _tc_body(phis_s, dists_s, sigma_s, dx_ref, dy_ref, g_ref, out_ref):
    dx = dx_ref[...]
    dy = dy_ref[...]
    r = jnp.sqrt(dx * dx + dy * dy + 1e-10)
    theta = jnp.arctan2(dy, dx)
    s = jnp.maximum(sigma_s[0], 1e-10)
    angs = []
    for p in range(NPHI):
        t = theta - phis_s[p] + math.pi
        t = t - jnp.floor(t / (2.0 * math.pi)) * (2.0 * math.pi)
        dphi = t - math.pi
        angs.append(jnp.exp(-0.5 * (dphi / SIG_PHI) ** 2))
    rads = []
    for d in range(NDIST):
        dr = (r - dists_s[d]) / s
        rads.append(jnp.exp(-0.5 * dr * dr))
    ang_t = jnp.stack(angs, axis=1)
    rad_t = jnp.stack(rads, axis=1)
    w4 = ang_t[:, :, None, :] * rad_t[:, None, :, :]
    w_t = w4.reshape(BN, PD, K)
    den = jnp.sum(w_t, axis=2, keepdims=True) + 1e-10
    wn = w_t / den

    ii = lax.broadcasted_iota(jnp.int32, (GRP, GRP), 0)
    jj = lax.broadcasted_iota(jnp.int32, (GRP, GRP), 1)
    eye = jnp.where(ii == jj, 1.0, 0.0).astype(jnp.float32)
    lhs5 = wn.reshape(NG, GRP, PD, 1, K) * eye.reshape(1, GRP, 1, GRP, 1)
    lhs = lhs5.reshape(NG, GRP * PD, GRP * K)
    rhs = g_ref[...].reshape(NG, GRP * K, A_OUT)
    outs = []
    for g in range(NG):
        outs.append(jnp.dot(lhs[g], rhs[g],
                            preferred_element_type=jnp.float32))
    out = jnp.stack(outs, axis=0)
    out_ref[...] = out.reshape(BN, PD, A_OUT)


_tc_agg = pl.pallas_call(
    _tc_body,
    grid=(N // BN,),
    in_specs=[
        pl.BlockSpec(memory_space=pltpu.SMEM),
        pl.BlockSpec(memory_space=pltpu.SMEM),
        pl.BlockSpec(memory_space=pltpu.SMEM),
        pl.BlockSpec((BN, K), lambda i: (i, 0)),
        pl.BlockSpec((BN, K), lambda i: (i, 0)),
        pl.BlockSpec((BN, K, A_OUT), lambda i: (i, 0, 0)),
    ],
    out_specs=pl.BlockSpec((BN, PD, A_OUT), lambda i: (i, 0, 0)),
    out_shape=jax.ShapeDtypeStruct((N, PD, A_OUT), jnp.float32),
)


def kernel(x, nh_idx, coords_rel, phis, dists, sigma, W_lin):
    x2d = x.reshape(N, A_IN)
    y = _prew(x2d, W_lin)
    y_pack = lax.bitcast_convert_type(y, jnp.int32)
    flat_idx = nh_idx.reshape(NK).astype(jnp.int32)
    g = _get_sc_gather()(y_pack, flat_idx)
    g_bf = lax.bitcast_convert_type(g, jnp.float32).reshape(N, K, A_OUT)
    dx = coords_rel[:, :, 0]
    dy = coords_rel[:, :, 1]
    out = _tc_agg(phis, dists, sigma, dx, dy, g_bf)
    return out.reshape(1, N, NPHI, NDIST, 1, A_OUT)
